# Initial kernel scaffold; baseline (speedup 1.0000x reference)
import jax
import jax.numpy as jnp
from jax.experimental import pallas as pl


def kernel(predicted_relevance, observed_relevance, item_popularity, k, rmax, head_tail_split):
    raise NotImplementedError("write your pallas kernel here")


# speedup vs baseline: 2.5049x; 2.5049x over previous
import jax
import jax.numpy as jnp
from jax import lax
from jax.experimental import pallas as pl
from jax.experimental.pallas import tpu as pltpu

_LMBDA = 0.4
_K = 10
_RMAX = 100
_BU = 8

_F32_ONE_BITS = 0x3F800001


def _count_ge(key, mid):
    return jnp.sum(jnp.where(key >= mid, 1.0, 0.0), axis=1, keepdims=True)


def _kth_largest_key(key, kth_f32, n_iters=31):
    b = key.shape[0]
    lo0 = jnp.full((b, 1), -2, jnp.int32)
    hi0 = jnp.full((b, 1), _F32_ONE_BITS, jnp.int32)

    def body(_, c):
        lo, hi = c
        mid = lax.shift_right_arithmetic(lo + hi, 1)
        ge = _count_ge(key, mid) >= kth_f32
        return jnp.where(ge, mid, lo), jnp.where(ge, hi, mid)

    lo, _ = lax.fori_loop(0, n_iters, body, (lo0, hi0))
    return lo


def _tie_index_threshold(eq, iota, need_f32, n_iters=15):
    b, n = eq.shape
    lo0 = jnp.full((b, 1), -1, jnp.int32)
    hi0 = jnp.full((b, 1), n - 1, jnp.int32)

    def body(_, c):
        lo, hi = c
        mid = lax.shift_right_arithmetic(lo + hi, 1)
        cnt = jnp.sum(jnp.where(eq & (iota <= mid), 1.0, 0.0), axis=1,
                      keepdims=True)
        ge = cnt >= need_f32
        return jnp.where(ge, lo, mid), jnp.where(ge, mid, hi)

    _, hi = lax.fori_loop(0, n_iters, body, (lo0, hi0))
    return hi


def _ishead_kernel(pop_ref, hts_ref, out_ref):
    pop = pop_ref[...]
    pn = pop / jnp.sum(pop)
    key = lax.bitcast_convert_type(pn, jnp.int32)
    split_f = hts_ref[0].astype(jnp.float32)
    v = _kth_largest_key(key, jnp.full((1, 1), 1.0, jnp.float32) * split_f)
    c_gt = jnp.sum(jnp.where(key > v, 1.0, 0.0), axis=1, keepdims=True)
    need = split_f - c_gt
    n = pop.shape[1]
    iota = lax.broadcasted_iota(jnp.int32, (1, n), 1)
    eq = key == v
    t = _tie_index_threshold(eq, iota, need)
    out_ref[...] = jnp.where((key > v) | (eq & (iota <= t)), 1.0, 0.0)


def _xquad_kernel(pred_ref, obs_ref, ish_ref, picks_ref, key_ref, base_ref):
    pred = pred_ref[...]
    obs = obs_ref[...]
    ish = ish_ref[...]
    b, n = pred.shape

    rmin = jnp.min(pred, axis=1, keepdims=True)
    rmx = jnp.max(pred, axis=1, keepdims=True)
    s = (pred - rmin) / (rmx - rmin)
    masked = obs > 0
    sm = jnp.where(masked, jnp.float32(-10000.0), s)

    obs_sum = jnp.sum(obs, axis=1, keepdims=True)
    ist = 1.0 - ish
    p_head = jnp.sum(obs * ish, axis=1, keepdims=True) / obs_sum
    p_tail = jnp.sum(obs * ist, axis=1, keepdims=True) / obs_sum

    key_ref[...] = jnp.where(masked, jnp.int32(-1),
                             lax.bitcast_convert_type(s, jnp.int32))
    key = key_ref[...]

    kth = jnp.full((b, 1), float(_RMAX), jnp.float32)
    v = _kth_largest_key(key, kth)
    c_gt = jnp.sum(jnp.where(key > v, 1.0, 0.0), axis=1, keepdims=True)
    need = float(_RMAX) - c_gt
    iota = lax.broadcasted_iota(jnp.int32, (b, n), 1)
    eq = key == v
    t = _tie_index_threshold(eq, iota, need)
    cand = (key > v) | (eq & (iota <= t))

    base_ref[...] = jnp.where(cand, jnp.float32(1.0 - _LMBDA) * sm,
                              jnp.float32(-jnp.inf))

    ch = jnp.ones((b, 1), jnp.float32)
    ct = jnp.ones((b, 1), jnp.float32)
    picks = []
    for _step in range(_K):
        w = p_head * (ish * ch) + p_tail * (ist * ct)
        comb = base_ref[...] + jnp.float32(_LMBDA) * w
        m = jnp.max(comb, axis=1, keepdims=True)
        idx = jnp.min(jnp.where(comb == m, iota, jnp.int32(n)), axis=1,
                      keepdims=True)
        onehot = iota == idx
        hsel = jnp.sum(jnp.where(onehot, ish, 0.0), axis=1, keepdims=True)
        ch = ch * (1.0 - hsel)
        ct = ct * hsel
        base_ref[...] = jnp.where(onehot, jnp.float32(-jnp.inf), base_ref[...])
        picks.append(idx)
    pad = jnp.zeros((b, 16 - _K), jnp.int32)
    picks_ref[...] = jnp.concatenate(picks + [pad], axis=1)


def kernel(predicted_relevance, observed_relevance, item_popularity, k, rmax,
           head_tail_split):
    u, n = predicted_relevance.shape

    pop2d = item_popularity.reshape(1, n)
    hts = jnp.asarray(head_tail_split, jnp.int32).reshape(1)
    is_head = pl.pallas_call(
        _ishead_kernel,
        grid=(1,),
        in_specs=[
            pl.BlockSpec((1, n), lambda i: (0, 0)),
            pl.BlockSpec(memory_space=pltpu.SMEM),
        ],
        out_specs=pl.BlockSpec((1, n), lambda i: (0, 0)),
        out_shape=jax.ShapeDtypeStruct((1, n), jnp.float32),
    )(pop2d, hts)

    picks = pl.pallas_call(
        _xquad_kernel,
        grid=(u // _BU,),
        in_specs=[
            pl.BlockSpec((_BU, n), lambda i: (i, 0)),
            pl.BlockSpec((_BU, n), lambda i: (i, 0)),
            pl.BlockSpec((1, n), lambda i: (0, 0)),
        ],
        out_specs=pl.BlockSpec((_BU, 16), lambda i: (i, 0)),
        out_shape=jax.ShapeDtypeStruct((u, 16), jnp.int32),
        scratch_shapes=[
            pltpu.VMEM((_BU, n), jnp.int32),
            pltpu.VMEM((_BU, n), jnp.float32),
        ],
    )(predicted_relevance, observed_relevance, is_head)

    items = picks[:, :_K]

    def per_user(items_row):
        def step(out, xs):
            t, item = xs
            out = out.at[item].set((k - (t + 1.0)) / k)
            return out, item

        out, _ = jax.lax.scan(step, jnp.zeros((n,), jnp.float32),
                              (jnp.arange(_K, dtype=jnp.float32), items_row))
        return out

    return jax.vmap(per_user)(items)
